# baseline (speedup 1.0000x reference)
import jax
import jax.numpy as jnp
from jax.experimental import pallas as pl
from jax.experimental.pallas import tpu as pltpu

_BM = 1024


def _mm_kernel(params_ref, xt_ref, txt_ref, memt_ref, out_t_ref, out_ref):
    inv_t = 1.0 / params_ref[0]
    m = memt_ref[...]
    xs = xt_ref[...] * inv_t
    txs = txt_ref[...] * inv_t
    dn = (((0,), (0,)), ((), ()))
    out_ref[...] = jax.lax.dot_general(
        m, xs, dn, preferred_element_type=jnp.float32)
    out_t_ref[...] = jax.lax.dot_general(
        m, txs, dn, preferred_element_type=jnp.float32)


@jax.jit
def kernel(image_features, transformed_image_features, indices, memory, params):
    del indices
    B, D = image_features.shape
    M = memory.shape[0]
    xt = image_features.T
    txt = transformed_image_features.T
    memt = memory.T
    grid = (pl.cdiv(M, _BM),)
    out_shape = jax.ShapeDtypeStruct((M, B), jnp.float32)
    out_t_p, out_p = pl.pallas_call(
        _mm_kernel,
        grid=grid,
        in_specs=[
            pl.BlockSpec(memory_space=pltpu.SMEM),
            pl.BlockSpec((D, B), lambda j: (0, 0)),
            pl.BlockSpec((D, B), lambda j: (0, 0)),
            pl.BlockSpec((D, _BM), lambda j: (0, j)),
        ],
        out_specs=[
            pl.BlockSpec((_BM, B), lambda j: (j, 0)),
            pl.BlockSpec((_BM, B), lambda j: (j, 0)),
        ],
        out_shape=[out_shape, out_shape],
        compiler_params=pltpu.CompilerParams(
            dimension_semantics=("arbitrary",),
        ),
    )(params, xt, txt, memt)
    return (out_t_p.T, out_p.T)


# speedup vs baseline: 1.0122x; 1.0122x over previous
import jax
import jax.numpy as jnp
from jax.experimental import pallas as pl
from jax.experimental.pallas import tpu as pltpu

_BM = 2048


def _mm_kernel(params_ref, xt_ref, txt_ref, memt_ref, out_t_ref, out_ref):
    inv_t = 1.0 / params_ref[0]
    m = memt_ref[...]
    xs = xt_ref[...] * inv_t
    txs = txt_ref[...] * inv_t
    dn = (((0,), (0,)), ((), ()))
    out_ref[...] = jax.lax.dot_general(
        m, xs, dn, preferred_element_type=jnp.float32)
    out_t_ref[...] = jax.lax.dot_general(
        m, txs, dn, preferred_element_type=jnp.float32)


@jax.jit
def kernel(image_features, transformed_image_features, indices, memory, params):
    del indices
    B, D = image_features.shape
    M = memory.shape[0]
    xt = image_features.T
    txt = transformed_image_features.T
    memt = memory.T
    grid = (pl.cdiv(M, _BM),)
    out_shape = jax.ShapeDtypeStruct((M, B), jnp.float32)
    out_t_p, out_p = pl.pallas_call(
        _mm_kernel,
        grid=grid,
        in_specs=[
            pl.BlockSpec(memory_space=pltpu.SMEM),
            pl.BlockSpec((D, B), lambda j: (0, 0)),
            pl.BlockSpec((D, B), lambda j: (0, 0)),
            pl.BlockSpec((D, _BM), lambda j: (0, j)),
        ],
        out_specs=[
            pl.BlockSpec((_BM, B), lambda j: (j, 0)),
            pl.BlockSpec((_BM, B), lambda j: (j, 0)),
        ],
        out_shape=[out_shape, out_shape],
        compiler_params=pltpu.CompilerParams(
            dimension_semantics=("arbitrary",),
        ),
    )(params, xt, txt, memt)
    return (out_t_p.T, out_p.T)
